# baseline (speedup 1.0000x reference)
import jax
import jax.numpy as jnp
from jax.experimental import pallas as pl
from jax.experimental.pallas import tpu as pltpu

_CLUSTERS = 300
_ITERS = 10
_CHUNK = 8192


def _dot(a, b, dims):
    return jax.lax.dot_general(a, b, (dims, ((), ())),
                               preferred_element_type=jnp.float32,
                               precision=jax.lax.Precision.DEFAULT)


def _split3(x):
    hi = x.astype(jnp.bfloat16)
    r1 = x - hi.astype(jnp.float32)
    mid = r1.astype(jnp.bfloat16)
    lo = (r1 - mid.astype(jnp.float32)).astype(jnp.bfloat16)
    return hi, mid, lo


def _kmeans_body(s_ref, s2_ref, m_ref, q_ref, b_ref, mo_ref,
                 s2x_s, sp_s, means_s, m2_s, sums_s, bins_s):
    it = pl.program_id(0)
    ch = pl.program_id(1)
    n_ch = pl.num_programs(1)
    rows = pl.ds(ch * _CHUNK, _CHUNK)
    colt = jax.lax.broadcasted_iota(jnp.int32, (_CLUSTERS, _CHUNK), 0)

    @pl.when(it == 0)
    def _():
        chunk = s_ref[...]
        s2x_s[rows, :] = chunk + chunk
        hi, mid, lo = _split3(chunk)
        sp_s[rows, pl.ds(0, 128)] = hi
        sp_s[rows, pl.ds(128, 128)] = mid
        sp_s[rows, pl.ds(256, 128)] = lo
        onecol = jax.lax.broadcasted_iota(jnp.int32, (_CHUNK, 128), 1) == 0
        sp_s[rows, pl.ds(384, 128)] = onecol.astype(jnp.bfloat16)

        @pl.when(ch == 0)
        def _():
            means_s[...] = m_ref[...]

    @pl.when(it < _ITERS)
    def _():
        @pl.when(ch == 0)
        def _():
            sums_s[...] = jnp.zeros_like(sums_s)
            bins_s[...] = jnp.zeros_like(bins_s)
            means = means_s[...]
            m2_s[...] = jnp.sum(means * means, axis=1, keepdims=True)

        csl = pl.ds(0, 1), pl.ds(ch * _CHUNK, _CHUNK)
        dots2t = _dot(means_s[...], s2x_s[rows, :], ((1,), (1,)))
        ndt = (s2_ref[csl] - dots2t) + m2_s[...]
        minv = jnp.min(ndt, axis=0, keepdims=True)
        cand = jnp.where(ndt == minv, colt, _CLUSTERS)
        buckets = jnp.min(cand, axis=0, keepdims=True)
        oht = (colt == buckets).astype(jnp.bfloat16)

        b_ref[csl] = buckets
        acc = _dot(oht, sp_s[rows, :], ((1,), (0,)))
        sums_s[...] += ((acc[:, 0:128] + acc[:, 128:256])
                        + acc[:, 256:384])
        bins_s[...] += acc[:, 384:385]

        @pl.when(ch == n_ch - 1)
        def _():
            bins = bins_s[...]
            zero = bins == 0.0
            binsc = jnp.where(zero, 1.0, bins)
            new_means = sums_s[...] / binsc
            means_s[...] = jnp.where(zero, means_s[...], new_means)

    @pl.when(it == _ITERS)
    def _():
        csl = pl.ds(0, 1), pl.ds(ch * _CHUNK, _CHUNK)
        buckets = b_ref[csl]
        oht = (colt == buckets).astype(jnp.bfloat16)
        mhi, mmid, mlo = _split3(means_s[...])
        q_ref[rows, :] = ((_dot(oht, mhi, ((0,), (0,)))
                           + _dot(oht, mmid, ((0,), (0,))))
                          + _dot(oht, mlo, ((0,), (0,))))

        @pl.when(ch == 0)
        def _():
            mo_ref[...] = means_s[...]


def kernel(emb):
    B, T, E = emb.shape
    n = B * T
    flat = emb.reshape(n, E)
    perm = jax.random.permutation(jax.random.key(42), n)[:_CLUSTERS]
    means0 = flat[perm]
    s2row = jnp.sum(flat * flat, axis=-1, keepdims=True).reshape(1, n)
    n_ch = n // _CHUNK

    quantized, buckets, means = pl.pallas_call(
        _kmeans_body,
        grid=(_ITERS + 1, n_ch),
        in_specs=[
            pl.BlockSpec((_CHUNK, E),
                         lambda it, ch: (jnp.where(it == 0, ch, 0), 0)),
            pl.BlockSpec((1, n), lambda it, ch: (0, 0)),
            pl.BlockSpec((_CLUSTERS, E), lambda it, ch: (0, 0)),
        ],
        out_specs=[
            pl.BlockSpec((n, E), lambda it, ch: (0, 0)),
            pl.BlockSpec((1, n), lambda it, ch: (0, 0)),
            pl.BlockSpec((_CLUSTERS, E), lambda it, ch: (0, 0)),
        ],
        out_shape=(
            jax.ShapeDtypeStruct((n, E), jnp.float32),
            jax.ShapeDtypeStruct((1, n), jnp.int32),
            jax.ShapeDtypeStruct((_CLUSTERS, E), jnp.float32),
        ),
        scratch_shapes=[
            pltpu.VMEM((n, E), jnp.float32),
            pltpu.VMEM((n, 512), jnp.bfloat16),
            pltpu.VMEM((_CLUSTERS, E), jnp.float32),
            pltpu.VMEM((_CLUSTERS, 1), jnp.float32),
            pltpu.VMEM((_CLUSTERS, E), jnp.float32),
            pltpu.VMEM((_CLUSTERS, 1), jnp.float32),
        ],
    )(flat, s2row, means0)

    tokens = buckets.reshape(B, T)
    return quantized.reshape(B, T, E), tokens, means


# speedup vs baseline: 1.0120x; 1.0120x over previous
import jax
import jax.numpy as jnp
from jax.experimental import pallas as pl
from jax.experimental.pallas import tpu as pltpu

_CLUSTERS = 300
_ITERS = 10
_CHUNK = 8192


def _dot(a, b, dims):
    return jax.lax.dot_general(a, b, (dims, ((), ())),
                               preferred_element_type=jnp.float32,
                               precision=jax.lax.Precision.DEFAULT)


def _split3(x):
    hi = x.astype(jnp.bfloat16)
    r1 = x - hi.astype(jnp.float32)
    mid = r1.astype(jnp.bfloat16)
    lo = (r1 - mid.astype(jnp.float32)).astype(jnp.bfloat16)
    return hi, mid, lo


def _kmeans_body(s_ref, s2_ref, m_ref, q_ref, b_ref, mo_ref,
                 s2x_s, sp_s, means_s, m2_s, sums_s, bins_s):
    it = pl.program_id(0)
    ch = pl.program_id(1)
    n_ch = pl.num_programs(1)
    rows = pl.ds(ch * _CHUNK, _CHUNK)
    colt = jax.lax.broadcasted_iota(jnp.int32, (_CLUSTERS, _CHUNK), 0)

    @pl.when(it == 0)
    def _():
        chunk = s_ref[...]
        s2x_s[rows, :] = chunk + chunk
        hi, mid, lo = _split3(chunk)
        sp_s[rows, pl.ds(0, 128)] = hi
        sp_s[rows, pl.ds(128, 128)] = mid
        sp_s[rows, pl.ds(256, 128)] = lo
        onecol = jax.lax.broadcasted_iota(jnp.int32, (_CHUNK, 128), 1) == 0
        sp_s[rows, pl.ds(384, 128)] = onecol.astype(jnp.bfloat16)

        @pl.when(ch == 0)
        def _():
            means_s[...] = m_ref[...]

    @pl.when(it < _ITERS)
    def _():
        @pl.when(ch == 0)
        def _():
            sums_s[...] = jnp.zeros_like(sums_s)
            bins_s[...] = jnp.zeros_like(bins_s)
            means = means_s[...]
            m2_s[...] = jnp.sum(means * means, axis=1, keepdims=True)

        nsub = 4
        sub = _CHUNK // nsub
        colsub = jax.lax.broadcasted_iota(jnp.int32, (_CLUSTERS, sub), 0)
        means = means_s[...]
        m2col = m2_s[...]
        srows = [pl.ds(ch * _CHUNK + i * sub, sub) for i in range(nsub)]
        scols = [(pl.ds(0, 1), pl.ds(ch * _CHUNK + i * sub, sub))
                 for i in range(nsub)]
        dots2t = [_dot(means, s2x_s[r, :], ((1,), (1,))) for r in srows]

        accs = []
        for i in range(nsub):
            ndt = (s2_ref[scols[i]] - dots2t[i]) + m2col
            minv = jnp.min(ndt, axis=0, keepdims=True)
            cand = jnp.where(ndt == minv, colsub, _CLUSTERS)
            buckets = jnp.min(cand, axis=0, keepdims=True)
            oht = (colsub == buckets).astype(jnp.bfloat16)
            b_ref[scols[i]] = buckets
            accs.append(_dot(oht, sp_s[srows[i], :], ((1,), (0,))))

        acc = (accs[0] + accs[1]) + (accs[2] + accs[3])
        sums_s[...] += ((acc[:, 0:128] + acc[:, 128:256])
                        + acc[:, 256:384])
        bins_s[...] += acc[:, 384:385]

        @pl.when(ch == n_ch - 1)
        def _():
            bins = bins_s[...]
            zero = bins == 0.0
            binsc = jnp.where(zero, 1.0, bins)
            new_means = sums_s[...] / binsc
            means_s[...] = jnp.where(zero, means_s[...], new_means)

    @pl.when(it == _ITERS)
    def _():
        csl = pl.ds(0, 1), pl.ds(ch * _CHUNK, _CHUNK)
        buckets = b_ref[csl]
        oht = (colt == buckets).astype(jnp.bfloat16)
        mhi, mmid, mlo = _split3(means_s[...])
        q_ref[rows, :] = ((_dot(oht, mhi, ((0,), (0,)))
                           + _dot(oht, mmid, ((0,), (0,))))
                          + _dot(oht, mlo, ((0,), (0,))))

        @pl.when(ch == 0)
        def _():
            mo_ref[...] = means_s[...]


def kernel(emb):
    B, T, E = emb.shape
    n = B * T
    flat = emb.reshape(n, E)
    perm = jax.random.permutation(jax.random.key(42), n)[:_CLUSTERS]
    means0 = flat[perm]
    s2row = jnp.sum(flat * flat, axis=-1, keepdims=True).reshape(1, n)
    n_ch = n // _CHUNK

    quantized, buckets, means = pl.pallas_call(
        _kmeans_body,
        grid=(_ITERS + 1, n_ch),
        in_specs=[
            pl.BlockSpec((_CHUNK, E),
                         lambda it, ch: (jnp.where(it == 0, ch, 0), 0)),
            pl.BlockSpec((1, n), lambda it, ch: (0, 0)),
            pl.BlockSpec((_CLUSTERS, E), lambda it, ch: (0, 0)),
        ],
        out_specs=[
            pl.BlockSpec((n, E), lambda it, ch: (0, 0)),
            pl.BlockSpec((1, n), lambda it, ch: (0, 0)),
            pl.BlockSpec((_CLUSTERS, E), lambda it, ch: (0, 0)),
        ],
        out_shape=(
            jax.ShapeDtypeStruct((n, E), jnp.float32),
            jax.ShapeDtypeStruct((1, n), jnp.int32),
            jax.ShapeDtypeStruct((_CLUSTERS, E), jnp.float32),
        ),
        scratch_shapes=[
            pltpu.VMEM((n, E), jnp.float32),
            pltpu.VMEM((n, 512), jnp.bfloat16),
            pltpu.VMEM((_CLUSTERS, E), jnp.float32),
            pltpu.VMEM((_CLUSTERS, 1), jnp.float32),
            pltpu.VMEM((_CLUSTERS, E), jnp.float32),
            pltpu.VMEM((_CLUSTERS, 1), jnp.float32),
        ],
    )(flat, s2row, means0)

    tokens = buckets.reshape(B, T)
    return quantized.reshape(B, T, E), tokens, means
